# baseline (speedup 1.0000x reference)
import functools

import jax
import jax.numpy as jnp
from jax import lax
from jax.experimental import pallas as pl
from jax.experimental.pallas import tpu as pltpu
from jax.experimental.pallas import tpu_sc as plsc

N = 50000
E = 800000
D = 64
H = 32
CH = 125
K = 5
ROWS = E // CH
NS = 16
NC = 2
RPT = ROWS // NS
RPT_DEG = ROWS // (NS * NC)
NPT = N // NS
BZ = CH

_mesh = plsc.VectorSubcoreMesh(core_axis_name="c", subcore_axis_name="s")
_sc_params = pltpu.CompilerParams(use_tc_tiling_on_sc=False)


def _zero_acc(rows0, acc, s):
    z16 = jnp.zeros((16,), jnp.float32)

    def _zrow(i, _):
        rows0[i, 0:16] = z16
        rows0[i, 16:32] = z16
        return 0

    lax.fori_loop(0, BZ, _zrow, 0)

    def _zcopy(t, _):
        pltpu.sync_copy(rows0, acc.at[pl.ds(s * NPT + t * BZ, BZ)])
        return 0

    lax.fori_loop(0, NPT // BZ, _zcopy, 0)


def _agg_pipeline(c, s, feat_hbm, src2_hbm, dst_hbm, out_hbm, sidx, didx, rows,
                  acc, gsem, ssem):
    _zero_acc(rows.at[0], acc, s)
    plsc.subcore_barrier()

    base = s * RPT
    nb = RPT // K

    def _gwait(p, j):
        pltpu.make_async_copy(feat_hbm.at[sidx.at[p, j]], rows.at[j], gsem).wait()

    def _fire_block(b):
        p = lax.rem(b, 2)
        r0 = base + b * K
        pltpu.sync_copy(src2_hbm.at[c, pl.ds(r0, K)], sidx.at[p])
        pltpu.sync_copy(dst_hbm.at[pl.ds(r0, K)], didx.at[p])
        for j in range(K):
            pltpu.async_copy(feat_hbm.at[sidx.at[p, j]], rows.at[j], gsem)

    _fire_block(0)

    def _outer(b, _):
        p = lax.rem(b, 2)
        for j in range(K):
            _gwait(p, j)
            pltpu.async_copy(rows.at[j], acc.at[didx.at[p, j]], ssem, add=True)
        q = lax.rem(b + 1, 2)
        r1 = base + (b + 1) * K

        @pl.when(b + 1 < nb)
        def _():
            pltpu.sync_copy(src2_hbm.at[c, pl.ds(r1, K)], sidx.at[q])
            pltpu.sync_copy(dst_hbm.at[pl.ds(r1, K)], didx.at[q])

        for j in range(K):
            pltpu.make_async_copy(rows.at[j], acc.at[didx.at[p, j]], ssem).wait()

            @pl.when(b + 1 < nb)
            def _():
                pltpu.async_copy(feat_hbm.at[sidx.at[q, j]], rows.at[j], gsem)

        return 0

    lax.fori_loop(0, nb, _outer, 0)
    plsc.subcore_barrier()
    pltpu.sync_copy(acc.at[pl.ds(s * NPT, NPT)], out_hbm.at[pl.ds(s * NPT, NPT), c])


_AGG_SCRATCH = [
    pltpu.VMEM((2, K, CH), jnp.int32),
    pltpu.VMEM((2, K, CH), jnp.int32),
    pltpu.VMEM((K, CH, H), jnp.float32),
    pltpu.MemorySpace.VMEM_SHARED((N, H), jnp.float32),
    pltpu.SemaphoreType.DMA,
    pltpu.SemaphoreType.DMA,
]


@functools.partial(
    pl.kernel,
    out_type=jax.ShapeDtypeStruct((N, NC, H), jnp.float32),
    mesh=_mesh,
    scratch_types=_AGG_SCRATCH,
    compiler_params=_sc_params,
)
def _agg(feat_hbm, src2_hbm, dst_hbm, out_hbm, sidx, didx, rows, acc, gsem, ssem):
    c = lax.axis_index("c")
    s = lax.axis_index("s")
    _agg_pipeline(c, s, feat_hbm, src2_hbm, dst_hbm, out_hbm, sidx, didx, rows,
                  acc, gsem, ssem)


@functools.partial(
    pl.kernel,
    out_type=[jax.ShapeDtypeStruct((N, NC, H), jnp.float32),
              jax.ShapeDtypeStruct((N, NC, H), jnp.float32)],
    mesh=_mesh,
    scratch_types=_AGG_SCRATCH,
    compiler_params=_sc_params,
)
def _deg_agg(feat_hbm, src2_hbm, dst_hbm, deg_hbm, out_hbm, sidx, didx, rows,
             acc, gsem, ssem):
    c = lax.axis_index("c")
    s = lax.axis_index("s")

    _zero_acc(rows.at[0], acc, s)
    plsc.subcore_barrier()

    o16 = jnp.ones((16,), jnp.float32)

    def _orow(i, _):
        rows[1, i, 0:16] = o16
        rows[1, i, 16:32] = o16
        return 0

    lax.fori_loop(0, BZ, _orow, 0)

    base = (c * NS + s) * RPT_DEG

    def _deg_outer(i, _):
        r0 = base + i * K
        pltpu.sync_copy(dst_hbm.at[pl.ds(r0, K)], didx.at[0])
        for j in range(K):
            pltpu.sync_copy(rows.at[1], acc.at[didx.at[0, j]], add=True)
        return 0

    lax.fori_loop(0, RPT_DEG // K, _deg_outer, 0)
    plsc.subcore_barrier()
    pltpu.sync_copy(acc.at[pl.ds(s * NPT, NPT)], deg_hbm.at[pl.ds(s * NPT, NPT), c])

    _agg_pipeline(c, s, feat_hbm, src2_hbm, dst_hbm, out_hbm, sidx, didx, rows,
                  acc, gsem, ssem)


KH = 10


@functools.partial(
    pl.kernel,
    out_type=jax.ShapeDtypeStruct((N, NC, 16), jnp.float32),
    mesh=_mesh,
    scratch_types=[
        pltpu.VMEM((2, KH, CH), jnp.int32),
        pltpu.VMEM((2, KH, CH), jnp.int32),
        pltpu.VMEM((KH, CH, 16), jnp.float32),
        pltpu.MemorySpace.VMEM_SHARED((N, 16), jnp.float32),
        pltpu.SemaphoreType.DMA,
        pltpu.SemaphoreType.DMA,
    ],
    compiler_params=_sc_params,
)
def _head_agg(feat_hbm, src_hbm, dst_hbm, out_hbm, sidx, didx, rows, acc,
              gsem, ssem):
    c = lax.axis_index("c")
    s = lax.axis_index("s")

    z16 = jnp.zeros((16,), jnp.float32)

    def _zrow(i, _):
        rows[0, i, :] = z16
        return 0

    lax.fori_loop(0, CH, _zrow, 0)

    def _zcopy(t, _):
        pltpu.sync_copy(rows.at[0], acc.at[pl.ds(s * NPT + t * BZ, BZ)])
        return 0

    lax.fori_loop(0, NPT // BZ, _zcopy, 0)
    plsc.subcore_barrier()

    base = (c * NS + s) * RPT_DEG
    nb = RPT_DEG // KH

    def _fire_block(b):
        p = lax.rem(b, 2)
        r0 = base + b * KH
        pltpu.sync_copy(src_hbm.at[pl.ds(r0, KH)], sidx.at[p])
        pltpu.sync_copy(dst_hbm.at[pl.ds(r0, KH)], didx.at[p])
        for j in range(KH):
            pltpu.async_copy(feat_hbm.at[sidx.at[p, j]], rows.at[j], gsem)

    _fire_block(0)

    def _outer(b, _):
        p = lax.rem(b, 2)
        for j in range(KH):
            pltpu.make_async_copy(feat_hbm.at[sidx.at[p, j]], rows.at[j],
                                  gsem).wait()
            pltpu.async_copy(rows.at[j], acc.at[didx.at[p, j]], ssem, add=True)
        q = lax.rem(b + 1, 2)
        r1 = base + (b + 1) * KH

        @pl.when(b + 1 < nb)
        def _():
            pltpu.sync_copy(src_hbm.at[pl.ds(r1, KH)], sidx.at[q])
            pltpu.sync_copy(dst_hbm.at[pl.ds(r1, KH)], didx.at[q])

        for j in range(KH):
            pltpu.make_async_copy(rows.at[j], acc.at[didx.at[p, j]], ssem).wait()

            @pl.when(b + 1 < nb)
            def _():
                pltpu.async_copy(feat_hbm.at[sidx.at[q, j]], rows.at[j], gsem)

        return 0

    lax.fori_loop(0, nb, _outer, 0)
    plsc.subcore_barrier()
    pltpu.sync_copy(acc.at[pl.ds(s * NPT, NPT)], out_hbm.at[pl.ds(s * NPT, NPT), c])


_BN = 5000


def _x_spec():
    return pl.BlockSpec((_BN, D), lambda i: (i, 0))


def _hp_spec():
    return pl.BlockSpec((_BN, 2 * 16), lambda i: (i, 0))


def _w_spec():
    return pl.BlockSpec((D, D), lambda i: (0, 0))


def _b_spec():
    return pl.BlockSpec((1, D), lambda i: (0, 0))


def _degv(dref):
    return jnp.maximum(dref[:, 0:1] + dref[:, H:H + 1], 1.0)


def _stage1_body(x_ref, a_ref, d_ref, wr_ref, wn_ref, b_ref, o_ref):
    m = a_ref[...] / _degv(d_ref)
    h = x_ref[...] @ wr_ref[...] + m @ wn_ref[...] + b_ref[...]
    o_ref[...] = jnp.maximum(h, 0.0)


_stage1 = pl.pallas_call(
    _stage1_body,
    grid=(N // _BN,),
    in_specs=[_x_spec(), _x_spec(), _x_spec(), _w_spec(), _w_spec(), _b_spec()],
    out_specs=_x_spec(),
    out_shape=jax.ShapeDtypeStruct((N, D), jnp.float32),
)


def _stage2_body(h_ref, a_ref, d_ref, cwr, cwn, cb, rwr, rwn, rb,
                 cpn, rpn, cnpn, co_ref, ro_ref, po_ref):
    h = h_ref[...]
    m = a_ref[...] / _degv(d_ref)
    cf = jnp.maximum(h @ cwr[...] + m @ cwn[...] + cb[...], 0.0)
    rf = jnp.maximum(h @ rwr[...] + m @ rwn[...] + rb[...], 0.0)
    co_ref[...] = cf
    ro_ref[...] = rf
    po_ref[...] = jnp.concatenate(
        [cf @ cpn[...], rf @ rpn[...], rf @ cnpn[...],
         jnp.zeros((_BN, 9), jnp.float32)], axis=1)


_stage2 = pl.pallas_call(
    _stage2_body,
    grid=(N // _BN,),
    in_specs=[_x_spec(), _x_spec(), _x_spec(),
              _w_spec(), _w_spec(), _b_spec(),
              _w_spec(), _w_spec(), _b_spec(),
              pl.BlockSpec((D, 2), lambda i: (0, 0)),
              pl.BlockSpec((D, 4), lambda i: (0, 0)),
              pl.BlockSpec((D, 1), lambda i: (0, 0))],
    out_specs=[_x_spec(), _x_spec(),
               pl.BlockSpec((_BN, 16), lambda i: (i, 0))],
    out_shape=[jax.ShapeDtypeStruct((N, D), jnp.float32),
               jax.ShapeDtypeStruct((N, D), jnp.float32),
               jax.ShapeDtypeStruct((N, 16), jnp.float32)],
)


def _stage3_body(cf_ref, rf_ref, hp_ref, d_ref, cpw, cpb, rpw, rpb, cnw, cnb,
                 sc_ref, cls_ref, reg_ref, cen_ref):
    dg = _degv(d_ref)
    hp = hp_ref[...]
    m = (hp[:, 0:16] + hp[:, 16:32]) / dg
    cf = cf_ref[...]
    rf = rf_ref[...]
    cls_ref[...] = cf @ cpw[...] + m[:, 0:2] + cpb[...]
    reg_ref[...] = (rf @ rpw[...] + m[:, 2:6] + rpb[...]) * sc_ref[...]
    cen_ref[...] = rf @ cnw[...] + m[:, 6:7] + cnb[...]


_stage3 = pl.pallas_call(
    _stage3_body,
    grid=(N // _BN,),
    in_specs=[_x_spec(), _x_spec(), _hp_spec(), _x_spec(),
              pl.BlockSpec((D, 2), lambda i: (0, 0)),
              pl.BlockSpec((1, 2), lambda i: (0, 0)),
              pl.BlockSpec((D, 4), lambda i: (0, 0)),
              pl.BlockSpec((1, 4), lambda i: (0, 0)),
              pl.BlockSpec((D, 1), lambda i: (0, 0)),
              pl.BlockSpec((1, 1), lambda i: (0, 0)),
              pl.BlockSpec((1, 1), lambda i: (0, 0))],
    out_specs=[pl.BlockSpec((_BN, 2), lambda i: (i, 0)),
               pl.BlockSpec((_BN, 4), lambda i: (i, 0)),
               pl.BlockSpec((_BN, 1), lambda i: (i, 0))],
    out_shape=[jax.ShapeDtypeStruct((N, 2), jnp.float32),
               jax.ShapeDtypeStruct((N, 4), jnp.float32),
               jax.ShapeDtypeStruct((N, 1), jnp.float32)],
)


def kernel(x, edge_index, stem_Wr, stem_Wn, stem_b, clsc_Wr, clsc_Wn, clsc_b,
           regc_Wr, regc_Wn, regc_b, clsp_Wr, clsp_Wn, clsp_b,
           regp_Wr, regp_Wn, regp_b, cenp_Wr, cenp_Wn, cenp_b, scales):
    src = edge_index[0].reshape(ROWS, CH)
    dst = edge_index[1].reshape(ROWS, CH)
    src2 = jnp.stack([src * 2, src * 2 + 1])

    degp, aggx = _deg_agg(x.reshape(NC * N, H), src2, dst)
    degp = degp.reshape(N, D)
    h = _stage1(x, aggx.reshape(N, D), degp, stem_Wr, stem_Wn,
                stem_b.reshape(1, D))

    aggh = _agg(h.reshape(NC * N, H), src2, dst)
    cls2, reg2, proj = _stage2(h, aggh.reshape(N, D), degp,
                               clsc_Wr, clsc_Wn, clsc_b.reshape(1, D),
                               regc_Wr, regc_Wn, regc_b.reshape(1, D),
                               clsp_Wn, regp_Wn, cenp_Wn)

    hp = _head_agg(proj, src, dst)

    cls_o, reg_o, cen_o = _stage3(
        cls2, reg2, hp.reshape(N, 32), degp,
        clsp_Wr, clsp_b.reshape(1, 2),
        regp_Wr, regp_b.reshape(1, 4),
        cenp_Wr, cenp_b.reshape(1, 1),
        scales[0].reshape(1, 1))

    return (cls_o.reshape(1, N, 2), reg_o.reshape(1, N, 4),
            cen_o.reshape(1, N, 1))


# speedup vs baseline: 1.4419x; 1.4419x over previous
import functools

import jax
import jax.numpy as jnp
from jax import lax
from jax.experimental import pallas as pl
from jax.experimental.pallas import tpu as pltpu
from jax.experimental.pallas import tpu_sc as plsc

N = 50000
E = 800000
D = 64
H = 32
CH = 125
K = 5
ROWS = E // CH
NS = 16
NC = 2
RPT = ROWS // NS
RPT_DEG = ROWS // (NS * NC)
NPT = N // NS
BZ = CH

_mesh = plsc.VectorSubcoreMesh(core_axis_name="c", subcore_axis_name="s")
_sc_params = pltpu.CompilerParams(use_tc_tiling_on_sc=False)


def _zero_acc(rows0, acc, s):
    z16 = jnp.zeros((16,), jnp.float32)

    def _zrow(i, _):
        rows0[i, 0:16] = z16
        rows0[i, 16:32] = z16
        return 0

    lax.fori_loop(0, BZ, _zrow, 0)

    def _zcopy(t, _):
        pltpu.sync_copy(rows0, acc.at[pl.ds(s * NPT + t * BZ, BZ)])
        return 0

    lax.fori_loop(0, NPT // BZ, _zcopy, 0)


def _agg_pipeline(c, s, feat_hbm, src2_hbm, dst_hbm, out_hbm, sidx, didx, rows,
                  acc, gsem, ssem):
    _zero_acc(rows.at[0], acc, s)
    plsc.subcore_barrier()

    base = s * RPT
    nb = RPT // K

    def _gwait(p, j):
        pltpu.make_async_copy(feat_hbm.at[sidx.at[p, j]], rows.at[j], gsem).wait()

    def _fire_block(b):
        p = lax.rem(b, 2)
        r0 = base + b * K
        pltpu.sync_copy(src2_hbm.at[c, pl.ds(r0, K)], sidx.at[p])
        pltpu.sync_copy(dst_hbm.at[pl.ds(r0, K)], didx.at[p])
        for j in range(K):
            pltpu.async_copy(feat_hbm.at[sidx.at[p, j]], rows.at[j], gsem)

    _fire_block(0)

    def _outer(b, _):
        p = lax.rem(b, 2)
        for j in range(K):
            _gwait(p, j)
            pltpu.async_copy(rows.at[j], acc.at[didx.at[p, j]], ssem, add=True)
        q = lax.rem(b + 1, 2)
        r1 = base + (b + 1) * K

        @pl.when(b + 1 < nb)
        def _():
            pltpu.sync_copy(src2_hbm.at[c, pl.ds(r1, K)], sidx.at[q])
            pltpu.sync_copy(dst_hbm.at[pl.ds(r1, K)], didx.at[q])

        for j in range(K):
            pltpu.make_async_copy(rows.at[j], acc.at[didx.at[p, j]], ssem).wait()

            @pl.when(b + 1 < nb)
            def _():
                pltpu.async_copy(feat_hbm.at[sidx.at[q, j]], rows.at[j], gsem)

        return 0

    lax.fori_loop(0, nb, _outer, 0)
    plsc.subcore_barrier()
    pltpu.sync_copy(acc.at[pl.ds(s * NPT, NPT)], out_hbm.at[c, pl.ds(s * NPT, NPT)])


_AGG_SCRATCH = [
    pltpu.VMEM((2, K, CH), jnp.int32),
    pltpu.VMEM((2, K, CH), jnp.int32),
    pltpu.VMEM((K, CH, H), jnp.float32),
    pltpu.MemorySpace.VMEM_SHARED((N, H), jnp.float32),
    pltpu.SemaphoreType.DMA,
    pltpu.SemaphoreType.DMA,
]


@functools.partial(
    pl.kernel,
    out_type=jax.ShapeDtypeStruct((NC, N, H), jnp.float32),
    mesh=_mesh,
    scratch_types=_AGG_SCRATCH,
    compiler_params=_sc_params,
)
def _agg(feat_hbm, src2_hbm, dst_hbm, out_hbm, sidx, didx, rows, acc, gsem, ssem):
    c = lax.axis_index("c")
    s = lax.axis_index("s")
    _agg_pipeline(c, s, feat_hbm, src2_hbm, dst_hbm, out_hbm, sidx, didx, rows,
                  acc, gsem, ssem)


@functools.partial(
    pl.kernel,
    out_type=[jax.ShapeDtypeStruct((NC, N, H), jnp.float32),
              jax.ShapeDtypeStruct((NC, N, H), jnp.float32)],
    mesh=_mesh,
    scratch_types=_AGG_SCRATCH,
    compiler_params=_sc_params,
)
def _deg_agg(feat_hbm, src2_hbm, dst_hbm, deg_hbm, out_hbm, sidx, didx, rows,
             acc, gsem, ssem):
    c = lax.axis_index("c")
    s = lax.axis_index("s")

    _zero_acc(rows.at[0], acc, s)
    plsc.subcore_barrier()

    o16 = jnp.ones((16,), jnp.float32)

    def _orow(i, _):
        rows[1, i, 0:16] = o16
        rows[1, i, 16:32] = o16
        return 0

    lax.fori_loop(0, BZ, _orow, 0)

    base = (c * NS + s) * RPT_DEG

    def _deg_outer(i, _):
        r0 = base + i * K
        pltpu.sync_copy(dst_hbm.at[pl.ds(r0, K)], didx.at[0])
        for j in range(K):
            pltpu.sync_copy(rows.at[1], acc.at[didx.at[0, j]], add=True)
        return 0

    lax.fori_loop(0, RPT_DEG // K, _deg_outer, 0)
    plsc.subcore_barrier()
    pltpu.sync_copy(acc.at[pl.ds(s * NPT, NPT)], deg_hbm.at[c, pl.ds(s * NPT, NPT)])

    _agg_pipeline(c, s, feat_hbm, src2_hbm, dst_hbm, out_hbm, sidx, didx, rows,
                  acc, gsem, ssem)


KH = 10


@functools.partial(
    pl.kernel,
    out_type=jax.ShapeDtypeStruct((NC, N, 16), jnp.float32),
    mesh=_mesh,
    scratch_types=[
        pltpu.VMEM((2, KH, CH), jnp.int32),
        pltpu.VMEM((2, KH, CH), jnp.int32),
        pltpu.VMEM((KH, CH, 16), jnp.float32),
        pltpu.MemorySpace.VMEM_SHARED((N, 16), jnp.float32),
        pltpu.SemaphoreType.DMA,
        pltpu.SemaphoreType.DMA,
    ],
    compiler_params=_sc_params,
)
def _head_agg(feat_hbm, src_hbm, dst_hbm, out_hbm, sidx, didx, rows, acc,
              gsem, ssem):
    c = lax.axis_index("c")
    s = lax.axis_index("s")

    z16 = jnp.zeros((16,), jnp.float32)

    def _zrow(i, _):
        rows[0, i, :] = z16
        return 0

    lax.fori_loop(0, CH, _zrow, 0)

    def _zcopy(t, _):
        pltpu.sync_copy(rows.at[0], acc.at[pl.ds(s * NPT + t * BZ, BZ)])
        return 0

    lax.fori_loop(0, NPT // BZ, _zcopy, 0)
    plsc.subcore_barrier()

    base = (c * NS + s) * RPT_DEG
    nb = RPT_DEG // KH

    def _fire_block(b):
        p = lax.rem(b, 2)
        r0 = base + b * KH
        pltpu.sync_copy(src_hbm.at[pl.ds(r0, KH)], sidx.at[p])
        pltpu.sync_copy(dst_hbm.at[pl.ds(r0, KH)], didx.at[p])
        for j in range(KH):
            pltpu.async_copy(feat_hbm.at[sidx.at[p, j]], rows.at[j], gsem)

    _fire_block(0)

    def _outer(b, _):
        p = lax.rem(b, 2)
        for j in range(KH):
            pltpu.make_async_copy(feat_hbm.at[sidx.at[p, j]], rows.at[j],
                                  gsem).wait()
            pltpu.async_copy(rows.at[j], acc.at[didx.at[p, j]], ssem, add=True)
        q = lax.rem(b + 1, 2)
        r1 = base + (b + 1) * KH

        @pl.when(b + 1 < nb)
        def _():
            pltpu.sync_copy(src_hbm.at[pl.ds(r1, KH)], sidx.at[q])
            pltpu.sync_copy(dst_hbm.at[pl.ds(r1, KH)], didx.at[q])

        for j in range(KH):
            pltpu.make_async_copy(rows.at[j], acc.at[didx.at[p, j]], ssem).wait()

            @pl.when(b + 1 < nb)
            def _():
                pltpu.async_copy(feat_hbm.at[sidx.at[q, j]], rows.at[j], gsem)

        return 0

    lax.fori_loop(0, nb, _outer, 0)
    plsc.subcore_barrier()
    pltpu.sync_copy(acc.at[pl.ds(s * NPT, NPT)], out_hbm.at[c, pl.ds(s * NPT, NPT)])


_BN = 5000


def _feat_spec():
    return pl.BlockSpec((NC, _BN, H), lambda i: (0, i, 0))


def _hp_spec():
    return pl.BlockSpec((NC, _BN, 16), lambda i: (0, i, 0))


def _x_spec():
    return pl.BlockSpec((_BN, D), lambda i: (i, 0))


def _w_spec():
    return pl.BlockSpec((D, D), lambda i: (0, 0))


def _b_spec():
    return pl.BlockSpec((1, D), lambda i: (0, 0))


def _cat(ref):
    return jnp.concatenate([ref[0], ref[1]], axis=1)


def _degv(dref):
    return jnp.maximum(dref[0, :, 0:1] + dref[1, :, 0:1], 1.0)


def _stage1_body(x_ref, a_ref, d_ref, wr_ref, wn_ref, b_ref, o_ref):
    m = _cat(a_ref) / _degv(d_ref)
    h = x_ref[...] @ wr_ref[...] + m @ wn_ref[...] + b_ref[...]
    o_ref[...] = jnp.maximum(h, 0.0)


_stage1 = pl.pallas_call(
    _stage1_body,
    grid=(N // _BN,),
    in_specs=[_x_spec(), _feat_spec(), _feat_spec(), _w_spec(), _w_spec(), _b_spec()],
    out_specs=_x_spec(),
    out_shape=jax.ShapeDtypeStruct((N, D), jnp.float32),
)


def _stage2_body(h_ref, a_ref, d_ref, cwr, cwn, cb, rwr, rwn, rb,
                 cpn, rpn, cnpn, co_ref, ro_ref, po_ref):
    h = h_ref[...]
    m = _cat(a_ref) / _degv(d_ref)
    cf = jnp.maximum(h @ cwr[...] + m @ cwn[...] + cb[...], 0.0)
    rf = jnp.maximum(h @ rwr[...] + m @ rwn[...] + rb[...], 0.0)
    co_ref[...] = cf
    ro_ref[...] = rf
    po_ref[...] = jnp.concatenate(
        [cf @ cpn[...], rf @ rpn[...], rf @ cnpn[...],
         jnp.zeros((_BN, 9), jnp.float32)], axis=1)


_stage2 = pl.pallas_call(
    _stage2_body,
    grid=(N // _BN,),
    in_specs=[_x_spec(), _feat_spec(), _feat_spec(),
              _w_spec(), _w_spec(), _b_spec(),
              _w_spec(), _w_spec(), _b_spec(),
              pl.BlockSpec((D, 2), lambda i: (0, 0)),
              pl.BlockSpec((D, 4), lambda i: (0, 0)),
              pl.BlockSpec((D, 1), lambda i: (0, 0))],
    out_specs=[_x_spec(), _x_spec(),
               pl.BlockSpec((_BN, 16), lambda i: (i, 0))],
    out_shape=[jax.ShapeDtypeStruct((N, D), jnp.float32),
               jax.ShapeDtypeStruct((N, D), jnp.float32),
               jax.ShapeDtypeStruct((N, 16), jnp.float32)],
)


def _stage3_body(cf_ref, rf_ref, hp_ref, d_ref, cpw, cpb, rpw, rpb, cnw, cnb,
                 sc_ref, cls_ref, reg_ref, cen_ref):
    dg = _degv(d_ref)
    m = (hp_ref[0] + hp_ref[1]) / dg
    cf = cf_ref[...]
    rf = rf_ref[...]
    cls_ref[...] = cf @ cpw[...] + m[:, 0:2] + cpb[...]
    reg_ref[...] = (rf @ rpw[...] + m[:, 2:6] + rpb[...]) * sc_ref[...]
    cen_ref[...] = rf @ cnw[...] + m[:, 6:7] + cnb[...]


_stage3 = pl.pallas_call(
    _stage3_body,
    grid=(N // _BN,),
    in_specs=[_x_spec(), _x_spec(), _hp_spec(), _feat_spec(),
              pl.BlockSpec((D, 2), lambda i: (0, 0)),
              pl.BlockSpec((1, 2), lambda i: (0, 0)),
              pl.BlockSpec((D, 4), lambda i: (0, 0)),
              pl.BlockSpec((1, 4), lambda i: (0, 0)),
              pl.BlockSpec((D, 1), lambda i: (0, 0)),
              pl.BlockSpec((1, 1), lambda i: (0, 0)),
              pl.BlockSpec((1, 1), lambda i: (0, 0))],
    out_specs=[pl.BlockSpec((_BN, 2), lambda i: (i, 0)),
               pl.BlockSpec((_BN, 4), lambda i: (i, 0)),
               pl.BlockSpec((_BN, 1), lambda i: (i, 0))],
    out_shape=[jax.ShapeDtypeStruct((N, 2), jnp.float32),
               jax.ShapeDtypeStruct((N, 4), jnp.float32),
               jax.ShapeDtypeStruct((N, 1), jnp.float32)],
)


def kernel(x, edge_index, stem_Wr, stem_Wn, stem_b, clsc_Wr, clsc_Wn, clsc_b,
           regc_Wr, regc_Wn, regc_b, clsp_Wr, clsp_Wn, clsp_b,
           regp_Wr, regp_Wn, regp_b, cenp_Wr, cenp_Wn, cenp_b, scales):
    src = edge_index[0].reshape(ROWS, CH)
    dst = edge_index[1].reshape(ROWS, CH)
    src2 = jnp.stack([src * 2, src * 2 + 1])

    degp, aggx = _deg_agg(x.reshape(NC * N, H), src2, dst)
    h = _stage1(x, aggx, degp, stem_Wr, stem_Wn, stem_b.reshape(1, D))

    aggh = _agg(h.reshape(NC * N, H), src2, dst)
    cls2, reg2, proj = _stage2(h, aggh, degp,
                               clsc_Wr, clsc_Wn, clsc_b.reshape(1, D),
                               regc_Wr, regc_Wn, regc_b.reshape(1, D),
                               clsp_Wn, regp_Wn, cenp_Wn)

    hp = _head_agg(proj, src, dst)

    cls_o, reg_o, cen_o = _stage3(
        cls2, reg2, hp, degp,
        clsp_Wr, clsp_b.reshape(1, 2),
        regp_Wr, regp_b.reshape(1, 4),
        cenp_Wr, cenp_b.reshape(1, 1),
        scales[0].reshape(1, 1))

    return (cls_o.reshape(1, N, 2), reg_o.reshape(1, N, 4),
            cen_o.reshape(1, N, 1))
